# baseline (speedup 1.0000x reference)
import jax
import jax.numpy as jnp
from jax.experimental import pallas as pl
from jax.experimental.pallas import tpu as pltpu

IN_FEATURES = 16
OUT_FEATURES = 32
PACK = 8
K_PACKED = PACK * IN_FEATURES
N_PACKED = PACK * OUT_FEATURES
ROW_TILE = 8192
NUM_CORES = 2


def _make_body(n_blocks, tb, half, n_rows):
    tb8 = tb // PACK

    def body(x_hbm, w_ref, b_ref, other_hbm, out_hbm,
             x_buf, o_buf, y_buf, wbig, sx, so, sy):
        p = pl.program_id(0)
        base = p * half

        wbig[...] = jnp.zeros((K_PACKED, N_PACKED), jnp.float32)
        for j in range(PACK):
            wbig[j * IN_FEATURES:(j + 1) * IN_FEATURES,
                 j * OUT_FEATURES:(j + 1) * OUT_FEATURES] = \
                w_ref[:, :OUT_FEATURES]

        ov = other_hbm.reshape(n_rows // PACK, PACK, OUT_FEATURES)
        yv = out_hbm.reshape(n_rows // PACK, 2, 128)

        def in_copies(i, slot):
            t0 = pl.multiple_of((base + i * tb) // PACK, 8)
            cs = [pltpu.make_async_copy(x_hbm.at[pl.ds(t0, tb8), :],
                                        x_buf.at[slot], sx.at[slot])]
            for s in range(PACK):
                cs.append(pltpu.make_async_copy(
                    ov.at[pl.ds(t0, tb8), s, :],
                    o_buf.at[slot, s], so.at[slot]))
            return cs

        def out_copy(i, slot):
            t0 = pl.multiple_of((base + i * tb) // PACK, 8)
            return pltpu.make_async_copy(y_buf.at[slot],
                                         yv.at[pl.ds(t0, tb8), :, :],
                                         sy.at[slot])

        b32 = b_ref[:, :OUT_FEATURES]

        for c in in_copies(0, 0):
            c.start()
        for i in range(n_blocks):
            slot = i % 2
            if i + 1 < n_blocks:
                for c in in_copies(i + 1, 1 - slot):
                    c.start()
            for c in in_copies(i, slot):
                c.wait()
            if i >= 2:
                out_copy(i - 2, slot).wait()
            v = jnp.dot(x_buf[slot], wbig[...],
                        preferred_element_type=jnp.float32)
            for s in range(PACK):
                part = jnp.maximum(
                    v[:, s * OUT_FEATURES:(s + 1) * OUT_FEATURES]
                    + b32 + o_buf[slot, s], 0.0)
                q = (s % 4) * OUT_FEATURES
                y_buf[slot, :, s // 4, q:q + OUT_FEATURES] = part
            out_copy(i, slot).start()
        for k in range(max(n_blocks - 2, 0), n_blocks):
            out_copy(k, k % 2).wait()

    return body


@jax.jit
def kernel(x, w_padded, b_padded, other):
    B = x.shape[0]
    half = B // NUM_CORES
    tb = min(ROW_TILE, half)
    while half % tb or tb % PACK:
        tb -= 1
    n_blocks = half // tb

    x_packed = x.reshape(B // PACK, K_PACKED)

    out4 = pl.pallas_call(
        _make_body(n_blocks, tb, half, B),
        out_shape=jax.ShapeDtypeStruct((B // 4, 128), jnp.float32),
        grid=(NUM_CORES,),
        in_specs=[
            pl.BlockSpec(memory_space=pl.ANY),
            pl.BlockSpec((IN_FEATURES, 128), lambda i: (0, 0)),
            pl.BlockSpec((1, 128), lambda i: (0, 0)),
            pl.BlockSpec(memory_space=pl.ANY),
        ],
        out_specs=pl.BlockSpec(memory_space=pl.ANY),
        scratch_shapes=[
            pltpu.VMEM((2, tb // PACK, K_PACKED), jnp.float32),
            pltpu.VMEM((2, PACK, tb // PACK, OUT_FEATURES), jnp.float32),
            pltpu.VMEM((2, tb // PACK, 2, 128), jnp.float32),
            pltpu.VMEM((K_PACKED, N_PACKED), jnp.float32),
            pltpu.SemaphoreType.DMA((2,)),
            pltpu.SemaphoreType.DMA((2,)),
            pltpu.SemaphoreType.DMA((2,)),
        ],
        compiler_params=pltpu.CompilerParams(
            dimension_semantics=("parallel",),
        ),
    )(x_packed, w_padded, b_padded, other)

    return out4.reshape(B, OUT_FEATURES)


# speedup vs baseline: 1.6613x; 1.6613x over previous
import jax
import jax.numpy as jnp
from jax.experimental import pallas as pl
from jax.experimental.pallas import tpu as pltpu

IN_FEATURES = 16
OUT_FEATURES = 32
CAT = IN_FEATURES + OUT_FEATURES
OUT_WIDE = 128
ROW_TILE = 8192
NUM_CORES = 2


def _make_body(n_blocks, tb, half):
    def body(c_hbm, w_ref, b_ref, out_hbm, c_buf, y_buf, sc, sy):
        p = pl.program_id(0)
        base = p * half

        def in_copy(i, slot):
            r0 = base + i * tb
            return pltpu.make_async_copy(c_hbm.at[pl.ds(r0, tb), :],
                                         c_buf.at[slot], sc.at[slot])

        def out_copy(i, slot):
            r0 = base + i * tb
            return pltpu.make_async_copy(y_buf.at[slot],
                                         out_hbm.at[pl.ds(r0, tb), :],
                                         sy.at[slot])

        in_copy(0, 0).start()
        for i in range(n_blocks):
            slot = i % 2
            if i + 1 < n_blocks:
                in_copy(i + 1, 1 - slot).start()
            in_copy(i, slot).wait()
            if i >= 2:
                out_copy(i - 2, slot).wait()
            c = c_buf[slot]
            v = jnp.dot(c[:, :IN_FEATURES], w_ref[...],
                        preferred_element_type=jnp.float32)
            o128 = jnp.pad(c[:, IN_FEATURES:CAT],
                           ((0, 0), (0, OUT_WIDE - OUT_FEATURES)))
            y_buf[slot] = jnp.maximum(v + b_ref[...] + o128, 0.0)
            out_copy(i, slot).start()
        for k in range(max(n_blocks - 2, 0), n_blocks):
            out_copy(k, k % 2).wait()

    return body


@jax.jit
def kernel(x, w_padded, b_padded, other):
    B = x.shape[0]
    half = B // NUM_CORES
    tb = min(ROW_TILE, half)
    while half % tb:
        tb -= 1
    n_blocks = half // tb

    combined = jnp.concatenate([x, other], axis=1)

    out_wide = pl.pallas_call(
        _make_body(n_blocks, tb, half),
        out_shape=jax.ShapeDtypeStruct((B, OUT_WIDE), jnp.float32),
        grid=(NUM_CORES,),
        in_specs=[
            pl.BlockSpec(memory_space=pl.ANY),
            pl.BlockSpec((IN_FEATURES, OUT_WIDE), lambda i: (0, 0)),
            pl.BlockSpec((1, OUT_WIDE), lambda i: (0, 0)),
        ],
        out_specs=pl.BlockSpec(memory_space=pl.ANY),
        scratch_shapes=[
            pltpu.VMEM((2, tb, CAT), jnp.float32),
            pltpu.VMEM((2, tb, OUT_WIDE), jnp.float32),
            pltpu.SemaphoreType.DMA((2,)),
            pltpu.SemaphoreType.DMA((2,)),
        ],
        compiler_params=pltpu.CompilerParams(
            dimension_semantics=("parallel",),
        ),
    )(combined, w_padded, b_padded)

    return out_wide[:, :OUT_FEATURES]
